# baseline (speedup 1.0000x reference)
import functools

import jax
import jax.numpy as jnp
from jax import lax
from jax.experimental import pallas as pl
from jax.experimental.pallas import tpu as pltpu
from jax.experimental.pallas import tpu_sc as plsc

VOCAB = 1000000
DIM = 64
MAXLEN = 200
BATCH = 4096
NIDX = BATCH * MAXLEN
LANES = 16
IDX_W = 100


@functools.cache
def _build():
    try:
        info = plsc.get_sparse_core_info()
        nc, ns = info.num_cores, info.num_subcores
    except Exception:
        nc, ns = 2, 16
    nw = nc * ns
    rows_per_w = NIDX // nw
    chunks = rows_per_w // MAXLEN

    group = 4
    groups = chunks // group

    mesh = plsc.VectorSubcoreMesh(core_axis_name="c", subcore_axis_name="s")

    @functools.partial(
        pl.kernel,
        out_type=jax.ShapeDtypeStruct((NIDX, DIM), jnp.float32),
        mesh=mesh,
        compiler_params=pltpu.CompilerParams(use_tc_tiling_on_sc=False),
        scratch_types=[
            pltpu.VMEM((2 * group, IDX_W), jnp.int32),
            pltpu.VMEM((MAXLEN, DIM), jnp.float32),
            pltpu.VMEM((MAXLEN, DIM), jnp.float32),
        ],
    )
    def k(tok_hbm, table_hbm, pos_hbm, out_hbm, idx_v, rows_v, pos_v):
        wid = lax.axis_index("s") * nc + lax.axis_index("c")
        base = wid * rows_per_w
        pltpu.sync_copy(pos_hbm, pos_v)

        def group_body(g, _):
            goff = base + g * group * MAXLEN
            irow = pl.multiple_of(goff // IDX_W, 8)
            pltpu.sync_copy(tok_hbm.at[pl.ds(irow, 2 * group)],
                            idx_v)
            for j in range(group):
                off = pl.multiple_of(goff + j * MAXLEN, 8)
                pltpu.sync_copy(table_hbm.at[idx_v.at[2 * j]],
                                rows_v.at[pl.ds(0, IDX_W)])
                pltpu.sync_copy(table_hbm.at[idx_v.at[2 * j + 1]],
                                rows_v.at[pl.ds(IDX_W, IDX_W)])

                def add_row(r, _):
                    for d in range(DIM // LANES):
                        sl = pl.ds(d * LANES, LANES)
                        rows_v[r, sl] = rows_v[r, sl] + pos_v[r, sl]
                    return 0

                lax.fori_loop(0, MAXLEN, add_row, 0)
                pltpu.sync_copy(rows_v, out_hbm.at[pl.ds(off, MAXLEN)])
            return 0

        lax.fori_loop(0, groups, group_body, 0)

    return k


def kernel(response_tokens, response_table, position_table):
    tok = response_tokens.reshape(NIDX // IDX_W, IDX_W).astype(jnp.int32)
    out = _build()(tok, response_table, position_table)
    return out.reshape(BATCH, MAXLEN, DIM)
